# baseline (speedup 1.0000x reference)
import functools

import jax
import jax.numpy as jnp
from jax import lax
from jax.experimental import pallas as pl
from jax.experimental.pallas import tpu as pltpu
from jax.experimental.pallas import tpu_sc as plsc

D_MODEL = 64
D_PAD = 128
SCALE_F = 8.0
NUM_CORES = 2
NUM_SUBCORES = 16
NUM_WORKERS = NUM_CORES * NUM_SUBCORES
LANES = 16
CHUNK = 512
NBUF = 3


def kernel(token_ids, table):
    batch_shape = token_ids.shape
    idx = token_ids.reshape(-1)
    num_ids = idx.shape[0]
    per_worker = num_ids // NUM_WORKERS
    n_chunks = per_worker // CHUNK
    assert per_worker * NUM_WORKERS == num_ids
    assert n_chunks * CHUNK == per_worker
    assert n_chunks >= NBUF

    mesh = plsc.VectorSubcoreMesh(core_axis_name="c", subcore_axis_name="s")

    @functools.partial(
        pl.kernel,
        mesh=mesh,
        out_type=jax.ShapeDtypeStruct((num_ids, D_PAD), jnp.float32),
        scratch_types=(
            [pltpu.VMEM((CHUNK,), jnp.int32)] * NBUF
            + [pltpu.VMEM((CHUNK, D_MODEL), jnp.float32)] * NBUF
            + [pltpu.SemaphoreType.DMA] * (2 * NBUF)
        ),
        compiler_params=pltpu.CompilerParams(use_tc_tiling_on_sc=False),
    )
    def gather_scale(table_hbm, idx_hbm, out_hbm, *scratch):
        idx_v = scratch[:NBUF]
        rows_v = scratch[NBUF:2 * NBUF]
        sem_g = scratch[2 * NBUF:3 * NBUF]
        sem_o = scratch[3 * NBUF:4 * NBUF]
        wid = lax.axis_index("s") * NUM_CORES + lax.axis_index("c")
        base0 = wid * per_worker

        def start_gather(j, b):
            base = base0 + j * CHUNK
            pltpu.sync_copy(idx_hbm.at[pl.ds(base, CHUNK)], idx_v[b])
            return pltpu.async_copy(table_hbm.at[idx_v[b]], rows_v[b],
                                    sem_g[b])

        gathers = [None] * NBUF
        outs = [None] * NBUF
        for j in range(NBUF - 1):
            gathers[j] = start_gather(j, j)

        for j in range(n_chunks):
            b = j % NBUF
            nxt = j + NBUF - 1
            if nxt < n_chunks:
                nb = nxt % NBUF
                if outs[nb] is not None:
                    outs[nb].wait()
                    outs[nb] = None
                gathers[nb] = start_gather(nxt, nb)
            gathers[b].wait()
            if outs[b] is not None:
                outs[b].wait()
                outs[b] = None

            @pl.loop(0, CHUNK, step=2)
            def _(r):
                for u in range(2):
                    for c in range(0, D_MODEL, LANES):
                        sl = (r + u, pl.ds(c, LANES))
                        rows_v[b].at[sl][...] = (
                            rows_v[b].at[sl][...] * SCALE_F
                        )

            base = base0 + j * CHUNK
            outs[b] = pltpu.async_copy(
                rows_v[b],
                out_hbm.at[pl.ds(base, CHUNK), pl.ds(0, D_MODEL)],
                sem_o[b],
            )

        for b in range(NBUF):
            if outs[b] is not None:
                outs[b].wait()

    out = gather_scale(table, idx)
    return out[:, :D_MODEL].reshape(*batch_shape, D_MODEL)
